# baseline (speedup 1.0000x reference)
import functools

import jax
import jax.numpy as jnp
from jax import lax
from jax.experimental import pallas as pl
from jax.experimental.pallas import tpu as pltpu
from jax.experimental.pallas import tpu_sc as plsc


def _emb_call(B, V, D, NC, NS):
    NW = NC * NS
    b_per_w = B // NW
    CH = 128
    n_ch = b_per_w // CH
    W = 2 * D
    mesh = plsc.VectorSubcoreMesh(core_axis_name="c", subcore_axis_name="s")

    @functools.partial(
        pl.kernel,
        mesh=mesh,
        out_type=jax.ShapeDtypeStruct((B, D), jnp.float32),
        scratch_types=[
            pltpu.VMEM((b_per_w,), jnp.int32),
            pltpu.VMEM((CH,), jnp.int32),
            pltpu.VMEM((CH, W), jnp.float32),
            pltpu.VMEM((CH, D), jnp.float32),
            pltpu.SemaphoreType.DMA,
        ],
        compiler_params=pltpu.CompilerParams(needs_layout_passes=False),
    )
    def emb(idx_hbm, tbl_hbm, out_hbm, idx_v, blk_v, rows_v, om_v, sem):
        wid = lax.axis_index("s") * NC + lax.axis_index("c")
        base = wid * b_per_w
        pltpu.sync_copy(idx_hbm.at[wid], idx_v)
        lane = lax.iota(jnp.int32, 16)

        def chunk(c, carry):
            for g in range(CH // 16):
                v = idx_v[pl.ds(c * CH + g * 16, 16)]
                blk_v[pl.ds(g * 16, 16)] = lax.shift_right_logical(v, 1)
            pltpu.async_copy(tbl_hbm.at[blk_v], rows_v, sem).wait()
            for g in range(CH // 16):
                v = idx_v[pl.ds(c * CH + g * 16, 16)]
                rows16 = lane + g * 16
                off = (v & 1) * D
                for q in range(D):
                    x = plsc.load_gather(rows_v, [rows16, off + q])
                    plsc.store_scatter(om_v, [rows16, lane * 0 + q], x)
            pltpu.sync_copy(om_v, out_hbm.at[pl.ds(base + c * CH, CH)])
            return carry

        lax.fori_loop(0, n_ch, chunk, 0)

    return emb


def kernel(batch, embedding_table):
    (B,) = batch.shape
    V, D = embedding_table.shape
    info = plsc.get_sparse_core_info()
    NC, NS = info.num_cores, info.num_subcores
    NW = NC * NS
    idx = batch.astype(jnp.int32).reshape(NW, B // NW)
    tbl2 = embedding_table.reshape(V // 2, 2 * D)
    return _emb_call(B, V, D, NC, NS)(idx, tbl2)


# speedup vs baseline: 1.8259x; 1.8259x over previous
import functools

import jax
import jax.numpy as jnp
from jax import lax
from jax.experimental import pallas as pl
from jax.experimental.pallas import tpu as pltpu
from jax.experimental.pallas import tpu_sc as plsc


def _emb_call(B, D, NC, NS):
    NW = NC * NS
    b_per_w = B // NW
    mesh = plsc.VectorSubcoreMesh(core_axis_name="c", subcore_axis_name="s")

    @functools.partial(
        pl.kernel,
        mesh=mesh,
        out_type=jax.ShapeDtypeStruct((B, D), jnp.float32),
        scratch_types=[
            pltpu.VMEM((b_per_w,), jnp.int32),
            pltpu.VMEM((b_per_w, D), jnp.float32),
            pltpu.SemaphoreType.DMA,
        ],
    )
    def emb(idx_hbm, table_hbm, out_hbm, idx_v, rows_v, sem):
        wid = lax.axis_index("s") * NC + lax.axis_index("c")
        base = wid * b_per_w
        pltpu.sync_copy(idx_hbm.at[wid], idx_v)

        def body(g, carry):
            vec = idx_v[pl.ds(g * 16, 16)]
            for b in range(16):
                pltpu.make_async_copy(
                    table_hbm.at[pl.ds(vec[b], 1)],
                    rows_v.at[pl.ds(g * 16 + b, 1)],
                    sem,
                ).start()
            return carry

        lax.fori_loop(0, b_per_w // 16, body, 0)
        pltpu.make_async_copy(table_hbm.at[pl.ds(0, b_per_w)], rows_v, sem).wait()
        pltpu.sync_copy(rows_v, out_hbm.at[pl.ds(base, b_per_w)])

    return emb


def kernel(batch, embedding_table):
    (B,) = batch.shape
    _, D = embedding_table.shape
    info = plsc.get_sparse_core_info()
    NC, NS = info.num_cores, info.num_subcores
    NW = NC * NS
    idx = batch.astype(jnp.int32).reshape(NW, B // NW)
    return _emb_call(B, D, NC, NS)(idx, embedding_table)
